# baseline (speedup 1.0000x reference)
import jax
import jax.numpy as jnp
from jax.experimental import pallas as pl

_N = 20000
_C = 85
_NCLS = 80
_LANES = 128
_NBLK = 157
_NPAD = _NBLK * _LANES
_BLK = 128

_YOLO_SIZE = 416.0
_MAX_OUT = 10
_SCORE_THR = 0.3
_IOU_THR = 0.5


def _decode_body(y_ref, boxes_ref, scores_ref, classes_ref):
    x = y_ref[...]
    b = x[:, 0:4] / _YOLO_SIZE
    b = jnp.clip(b, 0.0, 1.0)
    bx = b[:, 0:1]
    by = b[:, 1:2]
    bw = b[:, 2:3]
    bh = b[:, 3:4]
    x1 = jnp.clip(bx - bw * 0.5, 0.0, 1.0)
    y1 = jnp.clip(by - bh * 0.5, 0.0, 1.0)
    x2 = jnp.clip(bx + bw * 0.5, 0.0, 1.0)
    y2 = jnp.clip(by + bh * 0.5, 0.0, 1.0)
    boxes_ref[...] = jnp.concatenate([x1, y1, x2, y2], axis=1)

    probs = x[:, 5:_C]
    maxp = jnp.max(probs, axis=1, keepdims=True)
    lane = jax.lax.broadcasted_iota(jnp.int32, (_BLK, _NCLS), 1)
    cls = jnp.min(jnp.where(probs == maxp, lane, jnp.int32(_NCLS)), axis=1,
                  keepdims=True)
    scores_ref[...] = x[:, 4:5] * maxp
    classes_ref[...] = cls


def _nms_body(sc_ref, pl_ref, sel_i_ref, sel_s_ref):
    scores0 = sc_ref[...]
    scores0 = jnp.where(scores0 >= _SCORE_THR, scores0, -jnp.inf)
    x1 = pl_ref[0]
    y1 = pl_ref[1]
    x2 = pl_ref[2]
    y2 = pl_ref[3]
    area = (x2 - x1) * (y2 - y1)
    ridx = jax.lax.broadcasted_iota(jnp.int32, (_NBLK, _LANES), 0)
    cidx = jax.lax.broadcasted_iota(jnp.int32, (_NBLK, _LANES), 1)
    fidx = ridx * _LANES + cidx
    lane = jax.lax.broadcasted_iota(jnp.int32, (1, _LANES), 1)

    def step(k, carry):
        scores, sel_i, sel_s = carry
        m = jnp.max(scores)
        valid = m > -jnp.inf
        bi = jnp.min(jnp.where(scores == m, fidx, jnp.int32(2 ** 30)))
        hit = fidx == bi
        bx1 = jnp.sum(jnp.where(hit, x1, 0.0))
        by1 = jnp.sum(jnp.where(hit, y1, 0.0))
        bx2 = jnp.sum(jnp.where(hit, x2, 0.0))
        by2 = jnp.sum(jnp.where(hit, y2, 0.0))
        barea = (bx2 - bx1) * (by2 - by1)
        ix1 = jnp.maximum(bx1, x1)
        iy1 = jnp.maximum(by1, y1)
        ix2 = jnp.minimum(bx2, x2)
        iy2 = jnp.minimum(by2, y2)
        inter = jnp.maximum(ix2 - ix1, 0.0) * jnp.maximum(iy2 - iy1, 0.0)
        union = barea + area - inter
        pos = union > 0.0
        iou = jnp.where(pos, inter / jnp.where(pos, union, 1.0), 0.0)
        scores = jnp.where((iou > _IOU_THR) | hit, -jnp.inf, scores)
        sel_i = jnp.where(lane == k, jnp.where(valid, bi, -1), sel_i)
        sel_s = jnp.where(lane == k, jnp.where(valid, m, 0.0), sel_s)
        return scores, sel_i, sel_s

    init = (scores0,
            jnp.full((1, _LANES), -1, jnp.int32),
            jnp.zeros((1, _LANES), jnp.float32))
    _, sel_i, sel_s = jax.lax.fori_loop(0, _MAX_OUT, step, init)
    sel_i_ref[...] = sel_i
    sel_s_ref[...] = sel_s


def _decode_call(yp):
    return pl.pallas_call(
        _decode_body,
        grid=(_NBLK,),
        in_specs=[pl.BlockSpec((_BLK, _C), lambda i: (i, 0))],
        out_specs=[
            pl.BlockSpec((_BLK, 4), lambda i: (i, 0)),
            pl.BlockSpec((_BLK, 1), lambda i: (i, 0)),
            pl.BlockSpec((_BLK, 1), lambda i: (i, 0)),
        ],
        out_shape=[
            jax.ShapeDtypeStruct((_NPAD, 4), jnp.float32),
            jax.ShapeDtypeStruct((_NPAD, 1), jnp.float32),
            jax.ShapeDtypeStruct((_NPAD, 1), jnp.int32),
        ],
    )(yp)


def _nms_call(scores2d, planes):
    return pl.pallas_call(
        _nms_body,
        out_shape=[
            jax.ShapeDtypeStruct((1, _LANES), jnp.int32),
            jax.ShapeDtypeStruct((1, _LANES), jnp.float32),
        ],
    )(scores2d, planes)


def kernel(y_pred):
    flat = jnp.reshape(y_pred, (-1, y_pred.shape[-1]))
    yp = jnp.pad(flat, ((0, _NPAD - _N), (0, 0)))
    boxes_pad, scores_pad, classes_pad = _decode_call(yp)
    boxes = boxes_pad[:_N]
    box_scores = scores_pad[:_N]
    box_classes = classes_pad[:_N]
    scores2d = jnp.reshape(scores_pad, (_NBLK, _LANES))
    planes = jnp.reshape(boxes_pad.T, (4, _NBLK, _LANES))
    sel_i, sel_s = _nms_call(scores2d, planes)
    return boxes, box_scores, box_classes, sel_i[0, :_MAX_OUT], sel_s[0, :_MAX_OUT]


# speedup vs baseline: 1.6486x; 1.6486x over previous
import jax
import jax.numpy as jnp
from jax.experimental import pallas as pl
from jax.experimental.pallas import tpu as pltpu

_N = 20000
_C = 85
_NCLS = 80
_LANES = 128
_BLK = 128
_NBLK = 157

_YOLO_SIZE = 416.0
_MAX_OUT = 10
_SCORE_THR = 0.3
_IOU_THR = 0.5


def _body(y_ref, boxes_ref, scores_ref, classes_ref, sel_i_ref, sel_s_ref,
          x1_s, y1_s, x2_s, y2_s, sc_s):
    i = pl.program_id(0)
    x = y_ref[...]
    b = jnp.clip(x[:, 0:4] / _YOLO_SIZE, 0.0, 1.0)
    bx = b[:, 0:1]
    by = b[:, 1:2]
    bw = b[:, 2:3]
    bh = b[:, 3:4]
    x1 = jnp.clip(bx - bw * 0.5, 0.0, 1.0)
    y1 = jnp.clip(by - bh * 0.5, 0.0, 1.0)
    x2 = jnp.clip(bx + bw * 0.5, 0.0, 1.0)
    y2 = jnp.clip(by + bh * 0.5, 0.0, 1.0)
    boxes_blk = jnp.concatenate([x1, y1, x2, y2], axis=1)
    boxes_ref[...] = boxes_blk

    probs = x[:, 5:_C]
    maxp = jnp.max(probs, axis=1, keepdims=True)
    lane = jax.lax.broadcasted_iota(jnp.int32, (_BLK, _NCLS), 1)
    cls = jnp.min(jnp.where(probs == maxp, lane, jnp.int32(_NCLS)), axis=1,
                  keepdims=True)
    score = x[:, 4:5] * maxp
    scores_ref[...] = score
    classes_ref[...] = cls

    t5 = jnp.transpose(jnp.concatenate([boxes_blk, score], axis=1))
    x1_s[pl.ds(i, 1), :] = t5[0:1]
    y1_s[pl.ds(i, 1), :] = t5[1:2]
    x2_s[pl.ds(i, 1), :] = t5[2:3]
    y2_s[pl.ds(i, 1), :] = t5[3:4]
    sc_s[pl.ds(i, 1), :] = t5[4:5]

    @pl.when(i == _NBLK - 1)
    def _nms():
        ridx = jax.lax.broadcasted_iota(jnp.int32, (_NBLK, _LANES), 0)
        cidx = jax.lax.broadcasted_iota(jnp.int32, (_NBLK, _LANES), 1)
        fidx = ridx * _LANES + cidx
        ax1 = x1_s[...]
        ay1 = y1_s[...]
        ax2 = x2_s[...]
        ay2 = y2_s[...]
        sc = sc_s[...]
        scores0 = jnp.where((fidx < _N) & (sc >= _SCORE_THR), sc, -jnp.inf)
        area = (ax2 - ax1) * (ay2 - ay1)
        lane = jax.lax.broadcasted_iota(jnp.int32, (1, _LANES), 1)

        def step(k, carry):
            scores, sel_i, sel_s = carry
            m = jnp.max(scores)
            valid = m > -jnp.inf
            bi = jnp.min(jnp.where(scores == m, fidx, jnp.int32(2 ** 30)))
            hit = fidx == bi
            bx1 = jnp.sum(jnp.where(hit, ax1, 0.0))
            by1 = jnp.sum(jnp.where(hit, ay1, 0.0))
            bx2 = jnp.sum(jnp.where(hit, ax2, 0.0))
            by2 = jnp.sum(jnp.where(hit, ay2, 0.0))
            barea = (bx2 - bx1) * (by2 - by1)
            ix1 = jnp.maximum(bx1, ax1)
            iy1 = jnp.maximum(by1, ay1)
            ix2 = jnp.minimum(bx2, ax2)
            iy2 = jnp.minimum(by2, ay2)
            inter = jnp.maximum(ix2 - ix1, 0.0) * jnp.maximum(iy2 - iy1, 0.0)
            union = barea + area - inter
            pos = union > 0.0
            iou = jnp.where(pos, inter / jnp.where(pos, union, 1.0), 0.0)
            scores = jnp.where((iou > _IOU_THR) | hit, -jnp.inf, scores)
            sel_i = jnp.where(lane == k, jnp.where(valid, bi, -1), sel_i)
            sel_s = jnp.where(lane == k, jnp.where(valid, m, 0.0), sel_s)
            return scores, sel_i, sel_s

        init = (scores0,
                jnp.full((1, _LANES), -1, jnp.int32),
                jnp.zeros((1, _LANES), jnp.float32))
        _, sel_i, sel_s = jax.lax.fori_loop(0, _MAX_OUT, step, init)
        sel_i_ref[...] = sel_i
        sel_s_ref[...] = sel_s


def kernel(y_pred):
    flat = jnp.reshape(y_pred, (-1, y_pred.shape[-1]))
    boxes, box_scores, box_classes, sel_i, sel_s = pl.pallas_call(
        _body,
        grid=(_NBLK,),
        in_specs=[pl.BlockSpec((_BLK, _C), lambda i: (i, 0))],
        out_specs=[
            pl.BlockSpec((_BLK, 4), lambda i: (i, 0)),
            pl.BlockSpec((_BLK, 1), lambda i: (i, 0)),
            pl.BlockSpec((_BLK, 1), lambda i: (i, 0)),
            pl.BlockSpec((1, _LANES), lambda i: (0, 0)),
            pl.BlockSpec((1, _LANES), lambda i: (0, 0)),
        ],
        out_shape=[
            jax.ShapeDtypeStruct((_N, 4), jnp.float32),
            jax.ShapeDtypeStruct((_N, 1), jnp.float32),
            jax.ShapeDtypeStruct((_N, 1), jnp.int32),
            jax.ShapeDtypeStruct((1, _LANES), jnp.int32),
            jax.ShapeDtypeStruct((1, _LANES), jnp.float32),
        ],
        scratch_shapes=[pltpu.VMEM((_NBLK, _LANES), jnp.float32)] * 5,
    )(flat)
    return boxes, box_scores, box_classes, sel_i[0, :_MAX_OUT], sel_s[0, :_MAX_OUT]


# speedup vs baseline: 4.2415x; 2.5728x over previous
import jax
import jax.numpy as jnp
from jax.experimental import pallas as pl
from jax.experimental.pallas import tpu as pltpu

_N = 20000
_C = 85
_NCLS = 80
_BLK = 2560
_NBLK = 8

_YOLO_SIZE = 416.0
_MAX_OUT = 10
_SCORE_THR = 0.3
_IOU_THR = 0.5


def _body(y_ref, boxes_ref, scores_ref, classes_ref, sel_i_ref, sel_s_ref,
          x1_s, y1_s, x2_s, y2_s, sc_s):
    i = pl.program_id(0)
    xT = jnp.transpose(y_ref[...])
    b = jnp.clip(xT[0:4] / _YOLO_SIZE, 0.0, 1.0)
    bx = b[0:1]
    by = b[1:2]
    bw = b[2:3]
    bh = b[3:4]
    x1 = jnp.clip(bx - bw * 0.5, 0.0, 1.0)
    y1 = jnp.clip(by - bh * 0.5, 0.0, 1.0)
    x2 = jnp.clip(bx + bw * 0.5, 0.0, 1.0)
    y2 = jnp.clip(by + bh * 0.5, 0.0, 1.0)
    boxesT = jnp.concatenate([x1, y1, x2, y2], axis=0)
    boxes_ref[...] = jnp.transpose(boxesT)

    probs = xT[5:_C]
    maxp = jnp.max(probs, axis=0, keepdims=True)
    clsid = jax.lax.broadcasted_iota(jnp.int32, (_NCLS, _BLK), 0)
    cls = jnp.min(jnp.where(probs == maxp, clsid, jnp.int32(_NCLS)), axis=0,
                  keepdims=True)
    score = xT[4:5] * maxp
    scores_ref[...] = jnp.transpose(score)
    classes_ref[...] = jnp.transpose(cls)

    x1_s[pl.ds(i, 1), :] = x1
    y1_s[pl.ds(i, 1), :] = y1
    x2_s[pl.ds(i, 1), :] = x2
    y2_s[pl.ds(i, 1), :] = y2
    sc_s[pl.ds(i, 1), :] = score

    @pl.when(i == _NBLK - 1)
    def _nms():
        ridx = jax.lax.broadcasted_iota(jnp.int32, (_NBLK, _BLK), 0)
        cidx = jax.lax.broadcasted_iota(jnp.int32, (_NBLK, _BLK), 1)
        fidx = ridx * _BLK + cidx
        ax1 = x1_s[...]
        ay1 = y1_s[...]
        ax2 = x2_s[...]
        ay2 = y2_s[...]
        sc = sc_s[...]
        scores0 = jnp.where((fidx < _N) & (sc >= _SCORE_THR), sc, -jnp.inf)
        area = (ax2 - ax1) * (ay2 - ay1)
        lane = jax.lax.broadcasted_iota(jnp.int32, (1, _BLK), 1)

        def step(k, carry):
            scores, sel_i, sel_s = carry
            m = jnp.max(scores)
            valid = m > -jnp.inf
            bi = jnp.min(jnp.where(scores == m, fidx, jnp.int32(2 ** 30)))
            hit = fidx == bi
            bx1 = jnp.sum(jnp.where(hit, ax1, 0.0))
            by1 = jnp.sum(jnp.where(hit, ay1, 0.0))
            bx2 = jnp.sum(jnp.where(hit, ax2, 0.0))
            by2 = jnp.sum(jnp.where(hit, ay2, 0.0))
            barea = (bx2 - bx1) * (by2 - by1)
            ix1 = jnp.maximum(bx1, ax1)
            iy1 = jnp.maximum(by1, ay1)
            ix2 = jnp.minimum(bx2, ax2)
            iy2 = jnp.minimum(by2, ay2)
            inter = jnp.maximum(ix2 - ix1, 0.0) * jnp.maximum(iy2 - iy1, 0.0)
            union = barea + area - inter
            pos = union > 0.0
            iou = jnp.where(pos, inter / jnp.where(pos, union, 1.0), 0.0)
            scores = jnp.where((iou > _IOU_THR) | hit, -jnp.inf, scores)
            sel_i = jnp.where(lane == k, jnp.where(valid, bi, -1), sel_i)
            sel_s = jnp.where(lane == k, jnp.where(valid, m, 0.0), sel_s)
            return scores, sel_i, sel_s

        init = (scores0,
                jnp.full((1, _BLK), -1, jnp.int32),
                jnp.zeros((1, _BLK), jnp.float32))
        _, sel_i, sel_s = jax.lax.fori_loop(0, _MAX_OUT, step, init)
        sel_i_ref[...] = sel_i[:, 0:128]
        sel_s_ref[...] = sel_s[:, 0:128]


def kernel(y_pred):
    flat = jnp.reshape(y_pred, (-1, y_pred.shape[-1]))
    boxes, box_scores, box_classes, sel_i, sel_s = pl.pallas_call(
        _body,
        grid=(_NBLK,),
        in_specs=[pl.BlockSpec((_BLK, _C), lambda i: (i, 0))],
        out_specs=[
            pl.BlockSpec((_BLK, 4), lambda i: (i, 0)),
            pl.BlockSpec((_BLK, 1), lambda i: (i, 0)),
            pl.BlockSpec((_BLK, 1), lambda i: (i, 0)),
            pl.BlockSpec((1, 128), lambda i: (0, 0)),
            pl.BlockSpec((1, 128), lambda i: (0, 0)),
        ],
        out_shape=[
            jax.ShapeDtypeStruct((_N, 4), jnp.float32),
            jax.ShapeDtypeStruct((_N, 1), jnp.float32),
            jax.ShapeDtypeStruct((_N, 1), jnp.int32),
            jax.ShapeDtypeStruct((1, 128), jnp.int32),
            jax.ShapeDtypeStruct((1, 128), jnp.float32),
        ],
        scratch_shapes=[pltpu.VMEM((_NBLK, _BLK), jnp.float32)] * 5,
    )(flat)
    return boxes, box_scores, box_classes, sel_i[0, :_MAX_OUT], sel_s[0, :_MAX_OUT]


# speedup vs baseline: 5.0723x; 1.1959x over previous
import jax
import jax.numpy as jnp
from jax.experimental import pallas as pl
from jax.experimental.pallas import tpu as pltpu

_N = 20000
_C = 85
_NCLS = 80
_BLK = 2560
_NBLK = 8

_YOLO_SIZE = 416.0
_MAX_OUT = 10
_SCORE_THR = 0.3
_IOU_THR = 0.5


def _body(y_ref, boxes_ref, scores_ref, classes_ref, sel_i_ref, sel_s_ref,
          x1_s, y1_s, x2_s, y2_s, sc_s):
    i = pl.program_id(0)
    xT = jnp.transpose(y_ref[...])
    b = jnp.clip(xT[0:4] / _YOLO_SIZE, 0.0, 1.0)
    bx = b[0:1]
    by = b[1:2]
    bw = b[2:3]
    bh = b[3:4]
    x1 = jnp.clip(bx - bw * 0.5, 0.0, 1.0)
    y1 = jnp.clip(by - bh * 0.5, 0.0, 1.0)
    x2 = jnp.clip(bx + bw * 0.5, 0.0, 1.0)
    y2 = jnp.clip(by + bh * 0.5, 0.0, 1.0)
    boxesT = jnp.concatenate([x1, y1, x2, y2], axis=0)
    boxes_ref[...] = jnp.transpose(boxesT)

    probs = xT[5:_C]
    maxp = jnp.max(probs, axis=0, keepdims=True)
    clsid = jax.lax.broadcasted_iota(jnp.int32, (_NCLS, _BLK), 0)
    cls = jnp.min(jnp.where(probs == maxp, clsid, jnp.int32(_NCLS)), axis=0,
                  keepdims=True)
    score = xT[4:5] * maxp
    scores_ref[...] = jnp.reshape(score, (1, 1, _BLK))
    classes_ref[...] = jnp.reshape(cls, (1, 1, _BLK))

    x1_s[pl.ds(i, 1), :] = x1
    y1_s[pl.ds(i, 1), :] = y1
    x2_s[pl.ds(i, 1), :] = x2
    y2_s[pl.ds(i, 1), :] = y2
    sc_s[pl.ds(i, 1), :] = score

    @pl.when(i == _NBLK - 1)
    def _nms():
        ridx = jax.lax.broadcasted_iota(jnp.int32, (_NBLK, _BLK), 0)
        cidx = jax.lax.broadcasted_iota(jnp.int32, (_NBLK, _BLK), 1)
        fidx = ridx * _BLK + cidx
        ax1 = x1_s[...]
        ay1 = y1_s[...]
        ax2 = x2_s[...]
        ay2 = y2_s[...]
        sc = sc_s[...]
        scores0 = jnp.where((fidx < _N) & (sc >= _SCORE_THR), sc, -jnp.inf)
        area = (ax2 - ax1) * (ay2 - ay1)
        lane = jax.lax.broadcasted_iota(jnp.int32, (1, _BLK), 1)

        def step(k, carry):
            scores, sel_i, sel_s = carry
            m = jnp.max(scores)
            valid = m > -jnp.inf
            bi = jnp.min(jnp.where(scores == m, fidx, jnp.int32(2 ** 30)))
            hit = fidx == bi
            bx1 = jnp.sum(jnp.where(hit, ax1, 0.0))
            by1 = jnp.sum(jnp.where(hit, ay1, 0.0))
            bx2 = jnp.sum(jnp.where(hit, ax2, 0.0))
            by2 = jnp.sum(jnp.where(hit, ay2, 0.0))
            barea = (bx2 - bx1) * (by2 - by1)
            ix1 = jnp.maximum(bx1, ax1)
            iy1 = jnp.maximum(by1, ay1)
            ix2 = jnp.minimum(bx2, ax2)
            iy2 = jnp.minimum(by2, ay2)
            inter = jnp.maximum(ix2 - ix1, 0.0) * jnp.maximum(iy2 - iy1, 0.0)
            union = barea + area - inter
            pos = union > 0.0
            iou = jnp.where(pos, inter / jnp.where(pos, union, 1.0), 0.0)
            scores = jnp.where((iou > _IOU_THR) | hit, -jnp.inf, scores)
            sel_i = jnp.where(lane == k, jnp.where(valid, bi, -1), sel_i)
            sel_s = jnp.where(lane == k, jnp.where(valid, m, 0.0), sel_s)
            return scores, sel_i, sel_s

        init = (scores0,
                jnp.full((1, _BLK), -1, jnp.int32),
                jnp.zeros((1, _BLK), jnp.float32))
        _, sel_i, sel_s = jax.lax.fori_loop(0, _MAX_OUT, step, init)
        sel_i_ref[...] = sel_i[:, 0:128]
        sel_s_ref[...] = sel_s[:, 0:128]


def kernel(y_pred):
    flat = jnp.reshape(y_pred, (-1, y_pred.shape[-1]))
    boxes_w, scores_w, classes_w, sel_i, sel_s = pl.pallas_call(
        _body,
        grid=(_NBLK,),
        in_specs=[pl.BlockSpec((_BLK, _C), lambda i: (i, 0))],
        out_specs=[
            pl.BlockSpec((_BLK, 4), lambda i: (i, 0)),
            pl.BlockSpec((1, 1, _BLK), lambda i: (i, 0, 0)),
            pl.BlockSpec((1, 1, _BLK), lambda i: (i, 0, 0)),
            pl.BlockSpec((1, 128), lambda i: (0, 0)),
            pl.BlockSpec((1, 128), lambda i: (0, 0)),
        ],
        out_shape=[
            jax.ShapeDtypeStruct((_NBLK * _BLK, 4), jnp.float32),
            jax.ShapeDtypeStruct((_NBLK, 1, _BLK), jnp.float32),
            jax.ShapeDtypeStruct((_NBLK, 1, _BLK), jnp.int32),
            jax.ShapeDtypeStruct((1, 128), jnp.int32),
            jax.ShapeDtypeStruct((1, 128), jnp.float32),
        ],
        scratch_shapes=[pltpu.VMEM((_NBLK, _BLK), jnp.float32)] * 5,
    )(flat)
    boxes = boxes_w[:_N]
    box_scores = jnp.reshape(scores_w, (-1,))[:_N].reshape(_N, 1)
    box_classes = jnp.reshape(classes_w, (-1,))[:_N].reshape(_N, 1)
    return boxes, box_scores, box_classes, sel_i[0, :_MAX_OUT], sel_s[0, :_MAX_OUT]


# speedup vs baseline: 5.1216x; 1.0097x over previous
import jax
import jax.numpy as jnp
from jax.experimental import pallas as pl
from jax.experimental.pallas import tpu as pltpu

_N = 20000
_C = 85
_NCLS = 80
_BLK = 5120
_NBLK = 4

_YOLO_SIZE = 416.0
_MAX_OUT = 10
_SCORE_THR = 0.3
_IOU_THR = 0.5


def _body(y_ref, boxes_ref, scores_ref, classes_ref, sel_i_ref, sel_s_ref,
          x1_s, y1_s, x2_s, y2_s, sc_s):
    i = pl.program_id(0)
    xT = jnp.transpose(y_ref[...])
    b = jnp.clip(xT[0:4] / _YOLO_SIZE, 0.0, 1.0)
    bx = b[0:1]
    by = b[1:2]
    bw = b[2:3]
    bh = b[3:4]
    x1 = jnp.clip(bx - bw * 0.5, 0.0, 1.0)
    y1 = jnp.clip(by - bh * 0.5, 0.0, 1.0)
    x2 = jnp.clip(bx + bw * 0.5, 0.0, 1.0)
    y2 = jnp.clip(by + bh * 0.5, 0.0, 1.0)
    boxesT = jnp.concatenate([x1, y1, x2, y2], axis=0)
    boxes_ref[...] = jnp.transpose(boxesT)

    probs = xT[5:_C]
    maxp = jnp.max(probs, axis=0, keepdims=True)
    clsid = jax.lax.broadcasted_iota(jnp.int32, (_NCLS, _BLK), 0)
    cls = jnp.min(jnp.where(probs == maxp, clsid, jnp.int32(_NCLS)), axis=0,
                  keepdims=True)
    score = xT[4:5] * maxp
    scores_ref[...] = jnp.reshape(score, (1, 1, _BLK))
    classes_ref[...] = jnp.reshape(cls, (1, 1, _BLK))

    x1_s[pl.ds(i, 1), :] = x1
    y1_s[pl.ds(i, 1), :] = y1
    x2_s[pl.ds(i, 1), :] = x2
    y2_s[pl.ds(i, 1), :] = y2
    sc_s[pl.ds(i, 1), :] = score

    @pl.when(i == _NBLK - 1)
    def _nms():
        ridx = jax.lax.broadcasted_iota(jnp.int32, (_NBLK, _BLK), 0)
        cidx = jax.lax.broadcasted_iota(jnp.int32, (_NBLK, _BLK), 1)
        fidx = ridx * _BLK + cidx
        ax1 = x1_s[...]
        ay1 = y1_s[...]
        ax2 = x2_s[...]
        ay2 = y2_s[...]
        sc = sc_s[...]
        scores0 = jnp.where((fidx < _N) & (sc >= _SCORE_THR), sc, -jnp.inf)
        area = (ax2 - ax1) * (ay2 - ay1)
        lane = jax.lax.broadcasted_iota(jnp.int32, (1, _BLK), 1)

        def step(k, carry):
            scores, sel_i, sel_s = carry
            m = jnp.max(scores)
            valid = m > -jnp.inf
            bi = jnp.min(jnp.where(scores == m, fidx, jnp.int32(2 ** 30)))
            hit = fidx == bi
            bx1 = jnp.sum(jnp.where(hit, ax1, 0.0))
            by1 = jnp.sum(jnp.where(hit, ay1, 0.0))
            bx2 = jnp.sum(jnp.where(hit, ax2, 0.0))
            by2 = jnp.sum(jnp.where(hit, ay2, 0.0))
            barea = (bx2 - bx1) * (by2 - by1)
            ix1 = jnp.maximum(bx1, ax1)
            iy1 = jnp.maximum(by1, ay1)
            ix2 = jnp.minimum(bx2, ax2)
            iy2 = jnp.minimum(by2, ay2)
            inter = jnp.maximum(ix2 - ix1, 0.0) * jnp.maximum(iy2 - iy1, 0.0)
            union = barea + area - inter
            pos = union > 0.0
            iou = jnp.where(pos, inter / jnp.where(pos, union, 1.0), 0.0)
            scores = jnp.where((iou > _IOU_THR) | hit, -jnp.inf, scores)
            sel_i = jnp.where(lane == k, jnp.where(valid, bi, -1), sel_i)
            sel_s = jnp.where(lane == k, jnp.where(valid, m, 0.0), sel_s)
            return scores, sel_i, sel_s

        init = (scores0,
                jnp.full((1, _BLK), -1, jnp.int32),
                jnp.zeros((1, _BLK), jnp.float32))
        _, sel_i, sel_s = jax.lax.fori_loop(0, _MAX_OUT, step, init)
        sel_i_ref[...] = sel_i[:, 0:128]
        sel_s_ref[...] = sel_s[:, 0:128]


def kernel(y_pred):
    flat = jnp.reshape(y_pred, (-1, y_pred.shape[-1]))
    boxes_w, scores_w, classes_w, sel_i, sel_s = pl.pallas_call(
        _body,
        grid=(_NBLK,),
        in_specs=[pl.BlockSpec((_BLK, _C), lambda i: (i, 0))],
        out_specs=[
            pl.BlockSpec((_BLK, 4), lambda i: (i, 0)),
            pl.BlockSpec((1, 1, _BLK), lambda i: (i, 0, 0)),
            pl.BlockSpec((1, 1, _BLK), lambda i: (i, 0, 0)),
            pl.BlockSpec((1, 128), lambda i: (0, 0)),
            pl.BlockSpec((1, 128), lambda i: (0, 0)),
        ],
        out_shape=[
            jax.ShapeDtypeStruct((_NBLK * _BLK, 4), jnp.float32),
            jax.ShapeDtypeStruct((_NBLK, 1, _BLK), jnp.float32),
            jax.ShapeDtypeStruct((_NBLK, 1, _BLK), jnp.int32),
            jax.ShapeDtypeStruct((1, 128), jnp.int32),
            jax.ShapeDtypeStruct((1, 128), jnp.float32),
        ],
        scratch_shapes=[pltpu.VMEM((_NBLK, _BLK), jnp.float32)] * 5,
    )(flat)
    boxes = boxes_w[:_N]
    box_scores = jnp.reshape(scores_w, (-1,))[:_N].reshape(_N, 1)
    box_classes = jnp.reshape(classes_w, (-1,))[:_N].reshape(_N, 1)
    return boxes, box_scores, box_classes, sel_i[0, :_MAX_OUT], sel_s[0, :_MAX_OUT]
